# Initial kernel scaffold; baseline (speedup 1.0000x reference)
import jax
import jax.numpy as jnp
from jax.experimental import pallas as pl


def kernel(edge_index, emb_weight, is_CL):
    raise NotImplementedError("write your pallas kernel here")


# speedup vs baseline: 5.3851x; 5.3851x over previous
import functools

import jax
import jax.numpy as jnp
from jax import lax
from jax.experimental import pallas as pl
from jax.experimental.pallas import tpu as pltpu
from jax.experimental.pallas import tpu_sc as plsc

N = 10000
D = 128
E = 320000
NUM_LAYERS = 3

NC = 2
NS = 16
NW = NC * NS
C = 128
CH = 80
EPT = C * CH
EPAD = NW * EPT
NZ = 10240
RPT = NZ // NS

_sc_mesh = plsc.VectorSubcoreMesh(core_axis_name="c", subcore_axis_name="s")


def _deg_body(col_hbm, zfill_hbm, ones_hbm, deg_out, deg_acc, colidx, ones_v):
    c = lax.axis_index("c")
    s = lax.axis_index("s")
    wid = s * NC + c
    pltpu.sync_copy(col_hbm.at[wid], colidx)
    pltpu.sync_copy(ones_hbm, ones_v)
    pltpu.sync_copy(zfill_hbm, deg_acc.at[pl.ds(s * RPT, RPT)])
    plsc.subcore_barrier()

    def body(j, carry):
        pltpu.sync_copy(ones_v, deg_acc.at[colidx.at[j]], add=True)
        return carry

    lax.fori_loop(0, CH, body, 0)
    plsc.subcore_barrier()
    pltpu.sync_copy(deg_acc.at[pl.ds(s * RPT, RPT)],
                    deg_out.at[c, pl.ds(s * RPT, RPT)])


_deg_kernel = pl.kernel(
    _deg_body,
    out_type=jax.ShapeDtypeStruct((NC, NZ), jnp.float32),
    mesh=_sc_mesh,
    scratch_types=[
        pltpu.VMEM_SHARED((NZ,), jnp.float32),
        pltpu.VMEM((CH, C), jnp.int32),
        pltpu.VMEM((C,), jnp.float32),
    ],
)


def _layer_body(u_hbm, row_hbm, col_hbm, zfill_hbm, zp_out,
                zacc, rowidx, colidx, buf0, sem0):
    c = lax.axis_index("c")
    s = lax.axis_index("s")
    wid = s * NC + c
    pltpu.sync_copy(row_hbm.at[wid], rowidx)
    pltpu.sync_copy(col_hbm.at[wid], colidx)
    pltpu.sync_copy(zfill_hbm, zacc.at[pl.ds(s * RPT, RPT)])
    plsc.subcore_barrier()

    def body(j, carry):
        pltpu.async_copy(u_hbm.at[rowidx.at[j]], buf0, sem0).wait()
        pltpu.sync_copy(buf0, zacc.at[colidx.at[j]], add=True)
        return carry

    lax.fori_loop(0, CH, body, 0)
    plsc.subcore_barrier()
    pltpu.sync_copy(zacc.at[pl.ds(s * RPT, RPT)],
                    zp_out.at[c, pl.ds(s * RPT, RPT)])


_layer_kernel = pl.kernel(
    _layer_body,
    out_type=jax.ShapeDtypeStruct((NC, NZ, D), jnp.float32),
    mesh=_sc_mesh,
    scratch_types=[
        pltpu.VMEM_SHARED((NZ, D), jnp.float32),
        pltpu.VMEM((CH, C), jnp.int32),
        pltpu.VMEM((CH, C), jnp.int32),
        pltpu.VMEM((C, D), jnp.float32),
        pltpu.SemaphoreType.DMA,
    ],
)


_TB = 512


def _scale0_body(deg_ref, emb_ref, dis_ref, u0_ref):
    deg = deg_ref[0] + deg_ref[1]
    dis = jnp.where(deg > 0.0,
                    lax.rsqrt(jnp.maximum(deg, 1e-12)), 0.0)
    dis_ref[...] = dis
    u0_ref[...] = emb_ref[...] * dis


_scale0 = pl.pallas_call(
    _scale0_body,
    grid=(NZ // _TB,),
    in_specs=[
        pl.BlockSpec((NC, _TB, 1), lambda i: (0, i, 0)),
        pl.BlockSpec((_TB, D), lambda i: (i, 0)),
    ],
    out_specs=[
        pl.BlockSpec((_TB, 1), lambda i: (i, 0)),
        pl.BlockSpec((_TB, D), lambda i: (i, 0)),
    ],
    out_shape=[
        jax.ShapeDtypeStruct((NZ, 1), jnp.float32),
        jax.ShapeDtypeStruct((NZ, D), jnp.float32),
    ],
)


def _merge_first_body(zp_ref, dis_ref, u_ref, macc_ref):
    x = (zp_ref[0] + zp_ref[1]) * dis_ref[...]
    u_ref[...] = x * dis_ref[...]
    macc_ref[...] = x


def _merge_mid_body(zp_ref, dis_ref, macc_in_ref, u_ref, macc_ref):
    x = (zp_ref[0] + zp_ref[1]) * dis_ref[...]
    u_ref[...] = x * dis_ref[...]
    macc_ref[...] = macc_in_ref[...] + x


def _merge_last_body(zp_ref, dis_ref, macc_in_ref, out_ref):
    x = (zp_ref[0] + zp_ref[1]) * dis_ref[...]
    out_ref[...] = (macc_in_ref[...] + x) * (1.0 / NUM_LAYERS)


_zp_spec = pl.BlockSpec((NC, _TB, D), lambda i: (0, i, 0))
_dis_spec = pl.BlockSpec((_TB, 1), lambda i: (i, 0))
_nd_spec = pl.BlockSpec((_TB, D), lambda i: (i, 0))
_nd_shape = jax.ShapeDtypeStruct((NZ, D), jnp.float32)

_merge_first = pl.pallas_call(
    _merge_first_body,
    grid=(NZ // _TB,),
    in_specs=[_zp_spec, _dis_spec],
    out_specs=[_nd_spec, _nd_spec],
    out_shape=[_nd_shape, _nd_shape],
)

_merge_mid = pl.pallas_call(
    _merge_mid_body,
    grid=(NZ // _TB,),
    in_specs=[_zp_spec, _dis_spec, _nd_spec],
    out_specs=[_nd_spec, _nd_spec],
    out_shape=[_nd_shape, _nd_shape],
)

_merge_last = pl.pallas_call(
    _merge_last_body,
    grid=(NZ // _TB,),
    in_specs=[_zp_spec, _dis_spec, _nd_spec],
    out_specs=_nd_spec,
    out_shape=_nd_shape,
)


def kernel(edge_index, emb_weight, is_CL=False):
    del is_CL
    row = edge_index[0]
    col = edge_index[1]
    pad = EPAD - E
    padv = jnp.full((pad,), N, jnp.int32)
    rowp = jnp.concatenate([row, padv]).reshape(NW, CH, C)
    colp = jnp.concatenate([col, padv]).reshape(NW, CH, C)
    embp = jnp.pad(emb_weight, ((0, NZ - N), (0, 0)))
    zfill1 = jnp.zeros((RPT,), jnp.float32)
    zfillD = jnp.zeros((RPT, D), jnp.float32)
    ones = jnp.ones((C,), jnp.float32)

    deg_p = _deg_kernel(colp, zfill1, ones)
    dis, u = _scale0(deg_p.reshape(NC, NZ, 1), embp)

    zp = _layer_kernel(u, rowp, colp, zfillD)
    u, macc = _merge_first(zp, dis)
    zp = _layer_kernel(u, rowp, colp, zfillD)
    u, macc = _merge_mid(zp, dis, macc)
    zp = _layer_kernel(u, rowp, colp, zfillD)
    out = _merge_last(zp, dis, macc)
    return out[:N]


# speedup vs baseline: 5.9117x; 1.0978x over previous
import functools

import jax
import jax.numpy as jnp
from jax import lax
from jax.experimental import pallas as pl
from jax.experimental.pallas import tpu as pltpu
from jax.experimental.pallas import tpu_sc as plsc

N = 10000
D = 128
E = 320000
NUM_LAYERS = 3

NC = 2
NS = 16
NW = NC * NS
C = 128
CH = 80
EPT = C * CH
EPAD = NW * EPT
NZ = 10240
RPT = NZ // NS

_sc_mesh = plsc.VectorSubcoreMesh(core_axis_name="c", subcore_axis_name="s")


def _deg_body(col_hbm, zfill_hbm, ones_hbm, deg_out, deg_acc, colidx, ones_v):
    c = lax.axis_index("c")
    s = lax.axis_index("s")
    wid = s * NC + c
    pltpu.sync_copy(col_hbm.at[wid], colidx)
    pltpu.sync_copy(ones_hbm, ones_v)
    pltpu.sync_copy(zfill_hbm, deg_acc.at[pl.ds(s * RPT, RPT)])
    plsc.subcore_barrier()

    def body(j, carry):
        pltpu.sync_copy(ones_v, deg_acc.at[colidx.at[j]], add=True)
        return carry

    lax.fori_loop(0, CH, body, 0)
    plsc.subcore_barrier()
    pltpu.sync_copy(deg_acc.at[pl.ds(s * RPT, RPT)],
                    deg_out.at[c, pl.ds(s * RPT, RPT)])


_deg_kernel = pl.kernel(
    _deg_body,
    out_type=jax.ShapeDtypeStruct((NC, NZ), jnp.float32),
    mesh=_sc_mesh,
    scratch_types=[
        pltpu.VMEM_SHARED((NZ,), jnp.float32),
        pltpu.VMEM((CH, C), jnp.int32),
        pltpu.VMEM((C,), jnp.float32),
    ],
)


_K = 2
_QB = 16
_NB = CH // _QB


def _layer_body(u_hbm, row_hbm, col_hbm, zfill_hbm, zp_out,
                zacc, colidx, idxslots, idxsems, bufs, sems):
    c = lax.axis_index("c")
    s = lax.axis_index("s")
    wid = s * NC + c
    pltpu.sync_copy(col_hbm.at[wid], colidx)
    pltpu.sync_copy(zfill_hbm, zacc.at[pl.ds(s * RPT, RPT)])
    pltpu.async_copy(row_hbm.at[wid, 0], idxslots[0], idxsems[0])
    plsc.subcore_barrier()

    for blk in range(_NB):
        islot = idxslots[blk % 2]
        if blk + 1 < _NB:
            pltpu.async_copy(row_hbm.at[wid, blk + 1],
                             idxslots[(blk + 1) % 2], idxsems[(blk + 1) % 2])
        pltpu.make_async_copy(row_hbm.at[wid, blk],
                              islot, idxsems[blk % 2]).wait()
        for b in range(_K):
            pltpu.async_copy(u_hbm.at[islot.at[b]], bufs[b], sems[b])
        for q in range(_QB):
            b = q % _K
            pltpu.make_async_copy(u_hbm.at[islot.at[q]],
                                  bufs[b], sems[b]).wait()
            pltpu.sync_copy(bufs[b], zacc.at[colidx.at[blk * _QB + q]],
                            add=True)
            if q + _K < _QB:
                pltpu.async_copy(u_hbm.at[islot.at[q + _K]], bufs[b], sems[b])

    plsc.subcore_barrier()
    pltpu.sync_copy(zacc.at[pl.ds(s * RPT, RPT)],
                    zp_out.at[c, pl.ds(s * RPT, RPT)])


_layer_kernel = pl.kernel(
    _layer_body,
    out_type=jax.ShapeDtypeStruct((NC, NZ, D), jnp.float32),
    mesh=_sc_mesh,
    scratch_types=[
        pltpu.VMEM_SHARED((NZ, D), jnp.float32),
        pltpu.VMEM((CH, C), jnp.int32),
        [pltpu.VMEM((_QB, C), jnp.int32) for _ in range(2)],
        [pltpu.SemaphoreType.DMA for _ in range(2)],
        [pltpu.VMEM((C, D), jnp.float32) for _ in range(_K)],
        [pltpu.SemaphoreType.DMA for _ in range(_K)],
    ],
)


_TB = 512


def _scale0_body(deg_ref, emb_ref, dis_ref, u0_ref):
    deg = deg_ref[0] + deg_ref[1]
    dis = jnp.where(deg > 0.0,
                    lax.rsqrt(jnp.maximum(deg, 1e-12)), 0.0)
    dis_ref[...] = dis
    u0_ref[...] = emb_ref[...] * dis


_scale0 = pl.pallas_call(
    _scale0_body,
    grid=(NZ // _TB,),
    in_specs=[
        pl.BlockSpec((NC, _TB, 1), lambda i: (0, i, 0)),
        pl.BlockSpec((_TB, D), lambda i: (i, 0)),
    ],
    out_specs=[
        pl.BlockSpec((_TB, 1), lambda i: (i, 0)),
        pl.BlockSpec((_TB, D), lambda i: (i, 0)),
    ],
    out_shape=[
        jax.ShapeDtypeStruct((NZ, 1), jnp.float32),
        jax.ShapeDtypeStruct((NZ, D), jnp.float32),
    ],
)


def _merge_first_body(zp_ref, dis_ref, u_ref, macc_ref):
    x = (zp_ref[0] + zp_ref[1]) * dis_ref[...]
    u_ref[...] = x * dis_ref[...]
    macc_ref[...] = x


def _merge_mid_body(zp_ref, dis_ref, macc_in_ref, u_ref, macc_ref):
    x = (zp_ref[0] + zp_ref[1]) * dis_ref[...]
    u_ref[...] = x * dis_ref[...]
    macc_ref[...] = macc_in_ref[...] + x


def _merge_last_body(zp_ref, dis_ref, macc_in_ref, out_ref):
    x = (zp_ref[0] + zp_ref[1]) * dis_ref[...]
    out_ref[...] = (macc_in_ref[...] + x) * (1.0 / NUM_LAYERS)


_zp_spec = pl.BlockSpec((NC, _TB, D), lambda i: (0, i, 0))
_dis_spec = pl.BlockSpec((_TB, 1), lambda i: (i, 0))
_nd_spec = pl.BlockSpec((_TB, D), lambda i: (i, 0))
_nd_shape = jax.ShapeDtypeStruct((NZ, D), jnp.float32)

_merge_first = pl.pallas_call(
    _merge_first_body,
    grid=(NZ // _TB,),
    in_specs=[_zp_spec, _dis_spec],
    out_specs=[_nd_spec, _nd_spec],
    out_shape=[_nd_shape, _nd_shape],
)

_merge_mid = pl.pallas_call(
    _merge_mid_body,
    grid=(NZ // _TB,),
    in_specs=[_zp_spec, _dis_spec, _nd_spec],
    out_specs=[_nd_spec, _nd_spec],
    out_shape=[_nd_shape, _nd_shape],
)

_merge_last = pl.pallas_call(
    _merge_last_body,
    grid=(NZ // _TB,),
    in_specs=[_zp_spec, _dis_spec, _nd_spec],
    out_specs=_nd_spec,
    out_shape=_nd_shape,
)


def kernel(edge_index, emb_weight, is_CL=False):
    del is_CL
    row = edge_index[0]
    col = edge_index[1]
    pad = EPAD - E
    padv = jnp.full((pad,), N, jnp.int32)
    rowp = jnp.concatenate([row, padv]).reshape(NW, _NB, _QB, C)
    colp = jnp.concatenate([col, padv]).reshape(NW, CH, C)
    embp = jnp.pad(emb_weight, ((0, NZ - N), (0, 0)))
    zfill1 = jnp.zeros((RPT,), jnp.float32)
    zfillD = jnp.zeros((RPT, D), jnp.float32)
    ones = jnp.ones((C,), jnp.float32)

    deg_p = _deg_kernel(colp, zfill1, ones)
    dis, u = _scale0(deg_p.reshape(NC, NZ, 1), embp)

    zp = _layer_kernel(u, rowp, colp, zfillD)
    u, macc = _merge_first(zp, dis)
    zp = _layer_kernel(u, rowp, colp, zfillD)
    u, macc = _merge_mid(zp, dis, macc)
    zp = _layer_kernel(u, rowp, colp, zfillD)
    out = _merge_last(zp, dis, macc)
    return out[:N]
